# baseline (speedup 1.0000x reference)
import jax
import jax.numpy as jnp
from jax.experimental import pallas as pl
from jax.experimental.pallas import tpu as pltpu

_K = 128
_NQ = 20000
_NU = 10000
_B = 512


def _padr(x, m=_B):
    r = (-x.shape[0]) % m
    if r == 0:
        return x
    return jnp.pad(x, ((0, r),) + ((0, 0),) * (x.ndim - 1))


def _zpq_kernel(h_ref, w_ref, a_ref, z_ref, p_ref, q_ref):
    h = h_ref[...]
    w = w_ref[...]
    at = a_ref[...].T
    z = jnp.dot(h, w.T, preferred_element_type=jnp.float32)
    z_ref[...] = z
    p_ref[...] = jnp.dot(z, at[:_K], preferred_element_type=jnp.float32)
    q_ref[...] = jnp.dot(z, at[_K:], preferred_element_type=jnp.float32)


def _make_edge_a(E, src_lo, dst_lo):
    def k(ed_ref, p_ref, q_ref, e_ref, m_ref):
        pid = pl.program_id(0)

        @pl.when(pid == 0)
        def _():
            m_ref[...] = jnp.full(m_ref.shape, -jnp.inf, jnp.float32)

        def body(i, c):
            g = pid * _B + i
            s = ed_ref[0, i] - src_lo
            d = ed_ref[1, i] - dst_lo
            x = p_ref[pl.ds(s, 1), :] + q_ref[pl.ds(d, 1), :]
            ev = jnp.where(x > 0, x, 0.01 * x)
            ev = jnp.where(g < E, ev, -jnp.inf)
            e_ref[pl.ds(i, 1), :] = ev
            m_ref[pl.ds(d, 1), :] = jnp.maximum(m_ref[pl.ds(d, 1), :], ev)
            return c

        jax.lax.fori_loop(0, _B, body, 0)

    return k


def _make_edge_b(E, src_lo, dst_lo):
    def k(ed_ref, e_ref, m_ref, z_ref, s_ref, u_ref):
        pid = pl.program_id(0)

        @pl.when(pid == 0)
        def _():
            s_ref[...] = jnp.zeros(s_ref.shape, jnp.float32)
            u_ref[...] = jnp.zeros(u_ref.shape, jnp.float32)

        def body(i, c):
            g = pid * _B + i
            s = ed_ref[0, i] - src_lo
            d = ed_ref[1, i] - dst_lo
            ev = e_ref[pl.ds(i, 1), :]
            mv = m_ref[pl.ds(d, 1), :]
            ex = jnp.exp(ev - mv)
            ex = jnp.where(g < E, ex, 0.0)
            s_ref[pl.ds(d, 1), :] = s_ref[pl.ds(d, 1), :] + ex
            u_ref[pl.ds(d, 1), :] = u_ref[pl.ds(d, 1), :] + ex * z_ref[pl.ds(s, 1), :]
            return c

        jax.lax.fori_loop(0, _B, body, 0)

    return k


def _gat(h, edges, W, a, src_lo, src_n, dst_lo, dst_n):
    hp = _padr(h)
    npad = hp.shape[0]
    E = edges.shape[1]
    ep_n = E + ((-E) % _B)
    if ep_n > E:
        pad_col = jnp.array([[src_lo], [dst_lo]], jnp.int32)
        ep = jnp.concatenate([edges, jnp.tile(pad_col, (1, ep_n - E))], axis=1)
    else:
        ep = edges

    z, p, q = pl.pallas_call(
        _zpq_kernel,
        grid=(npad // _B,),
        in_specs=[
            pl.BlockSpec((_B, _K), lambda i: (i, 0)),
            pl.BlockSpec((_K, _K), lambda i: (0, 0)),
            pl.BlockSpec((1, 2 * _K), lambda i: (0, 0)),
        ],
        out_specs=[
            pl.BlockSpec((_B, _K), lambda i: (i, 0)),
            pl.BlockSpec((_B, 1), lambda i: (i, 0)),
            pl.BlockSpec((_B, 1), lambda i: (i, 0)),
        ],
        out_shape=[
            jax.ShapeDtypeStruct((npad, _K), jnp.float32),
            jax.ShapeDtypeStruct((npad, 1), jnp.float32),
            jax.ShapeDtypeStruct((npad, 1), jnp.float32),
        ],
    )(hp, W, a)

    zs = _padr(z[src_lo:src_lo + src_n])
    ps = _padr(p[src_lo:src_lo + src_n])
    qd = _padr(q[dst_lo:dst_lo + dst_n])
    sp = zs.shape[0]
    dp = qd.shape[0]

    e, m = pl.pallas_call(
        _make_edge_a(E, src_lo, dst_lo),
        grid=(ep_n // _B,),
        in_specs=[
            pl.BlockSpec((2, _B), lambda i: (0, i), memory_space=pltpu.SMEM),
            pl.BlockSpec((sp, 1), lambda i: (0, 0)),
            pl.BlockSpec((dp, 1), lambda i: (0, 0)),
        ],
        out_specs=[
            pl.BlockSpec((_B, 1), lambda i: (i, 0)),
            pl.BlockSpec((dp, 1), lambda i: (0, 0)),
        ],
        out_shape=[
            jax.ShapeDtypeStruct((ep_n, 1), jnp.float32),
            jax.ShapeDtypeStruct((dp, 1), jnp.float32),
        ],
    )(ep, ps, qd)

    ssum, U = pl.pallas_call(
        _make_edge_b(E, src_lo, dst_lo),
        grid=(ep_n // _B,),
        in_specs=[
            pl.BlockSpec((2, _B), lambda i: (0, i), memory_space=pltpu.SMEM),
            pl.BlockSpec((_B, 1), lambda i: (i, 0)),
            pl.BlockSpec((dp, 1), lambda i: (0, 0)),
            pl.BlockSpec((sp, _K), lambda i: (0, 0)),
        ],
        out_specs=[
            pl.BlockSpec((dp, 1), lambda i: (0, 0)),
            pl.BlockSpec((dp, _K), lambda i: (0, 0)),
        ],
        out_shape=[
            jax.ShapeDtypeStruct((dp, 1), jnp.float32),
            jax.ShapeDtypeStruct((dp, _K), jnp.float32),
        ],
    )(ep, e, m, zs)
    return U, ssum


def _fuse_kn_kernel(a_ref, ub_ref, sb_ref, uc_ref, sc_ref, ud_ref, sd_ref,
                    w1_ref, b1_ref, w2_ref, b2_ref, w3_ref, b3_ref, o_ref):
    A = a_ref[...]
    B = ub_ref[...] / (sb_ref[...] + 1e-9)
    C = uc_ref[...] / (sc_ref[...] + 1e-9)
    D = ud_ref[...] / (sd_ref[...] + 1e-9)
    w1 = w1_ref[...].T
    w2 = w2_ref[...].T
    w3 = w3_ref[...].T
    s1 = jnp.dot(A, w1[:_K]) + jnp.dot(B, w1[_K:]) + b1_ref[...]
    s2 = jnp.dot(A, w2[:_K]) + jnp.dot(C, w2[_K:]) + b2_ref[...]
    s3 = jnp.dot(A, w3[:_K]) + jnp.dot(D, w3[_K:]) + b3_ref[...]
    mx = jnp.maximum(jnp.maximum(s1, s2), s3)
    e1 = jnp.exp(s1 - mx)
    e2 = jnp.exp(s2 - mx)
    e3 = jnp.exp(s3 - mx)
    den = e1 + e2 + e3
    o_ref[...] = A + (e1 / den) * B + (e2 / den) * C + (e3 / den) * D


def _fuse_exer_kernel(a_ref, ub_ref, sb_ref, uc_ref, sc_ref,
                      w1_ref, b1_ref, w2_ref, b2_ref, o_ref):
    A = a_ref[...]
    B = ub_ref[...] / (sb_ref[...] + 1e-9)
    C = uc_ref[...] / (sc_ref[...] + 1e-9)
    w1 = w1_ref[...].T
    w2 = w2_ref[...].T
    t1 = jnp.dot(A, w1[:_K]) + jnp.dot(B, w1[_K:]) + b1_ref[...]
    t2 = jnp.dot(A, w2[:_K]) + jnp.dot(C, w2[_K:]) + b2_ref[...]
    mx = jnp.maximum(t1, t2)
    e1 = jnp.exp(t1 - mx)
    e2 = jnp.exp(t2 - mx)
    den = e1 + e2
    o_ref[...] = A + (e1 / den) * B + (e2 / den) * C


def _fuse_stu_kernel(a_ref, u_ref, s_ref, o_ref):
    o_ref[...] = a_ref[...] + u_ref[...] / (s_ref[...] + 1e-9)


def kernel(kn_emb, exer_emb, all_stu_emb, dir_edges, undir_edges, ke_edges,
           ek_edges, ue_edges, eu_edges, W_dir, a_dir, W_undir, a_undir,
           W_kfe, a_kfe, W_efk, a_efk, W_ufe, a_ufe, W_efu, a_efu,
           kw1, kb1, kw2, kb2, kw3, kb3, ew1, eb1, ew2, eb2):
    U_dir, s_dir = _gat(kn_emb, dir_edges, W_dir, a_dir, 0, _K, 0, _K)
    U_und, s_und = _gat(kn_emb, undir_edges, W_undir, a_undir, 0, _K, 0, _K)
    e_k = jnp.concatenate([exer_emb, kn_emb], axis=0)
    U_ke, s_ke = _gat(e_k, ke_edges, W_kfe, a_kfe, 0, _NQ, _NQ, _K)
    U_ek, s_ek = _gat(e_k, ek_edges, W_efk, a_efk, _NQ, _K, 0, _NQ)
    e_u = jnp.concatenate([exer_emb, all_stu_emb], axis=0)
    U_ue, s_ue = _gat(e_u, ue_edges, W_ufe, a_ufe, 0, _NQ, _NQ, _NU)
    U_eu, s_eu = _gat(e_u, eu_edges, W_efu, a_efu, _NQ, _NU, 0, _NQ)

    kb1r = kb1.reshape(1, 1)
    kb2r = kb2.reshape(1, 1)
    kb3r = kb3.reshape(1, 1)
    kn_out = pl.pallas_call(
        _fuse_kn_kernel,
        out_shape=jax.ShapeDtypeStruct((_K, _K), jnp.float32),
    )(kn_emb, U_dir[:_K], s_dir[:_K], U_und[:_K], s_und[:_K],
      U_ke[:_K], s_ke[:_K],
      kw1, kb1r, kw2, kb2r, kw3, kb3r)

    nqp = _NQ + ((-_NQ) % _B)
    a2 = _padr(exer_emb)
    ub2 = _padr(U_ek[:_NQ])
    sb2 = _padr(s_ek[:_NQ])
    uc2 = _padr(U_eu[:_NQ])
    sc2 = _padr(s_eu[:_NQ])
    exer_out = pl.pallas_call(
        _fuse_exer_kernel,
        grid=(nqp // _B,),
        in_specs=[
            pl.BlockSpec((_B, _K), lambda i: (i, 0)),
            pl.BlockSpec((_B, _K), lambda i: (i, 0)),
            pl.BlockSpec((_B, 1), lambda i: (i, 0)),
            pl.BlockSpec((_B, _K), lambda i: (i, 0)),
            pl.BlockSpec((_B, 1), lambda i: (i, 0)),
            pl.BlockSpec((1, 2 * _K), lambda i: (0, 0)),
            pl.BlockSpec((1, 1), lambda i: (0, 0)),
            pl.BlockSpec((1, 2 * _K), lambda i: (0, 0)),
            pl.BlockSpec((1, 1), lambda i: (0, 0)),
        ],
        out_specs=pl.BlockSpec((_B, _K), lambda i: (i, 0)),
        out_shape=jax.ShapeDtypeStruct((nqp, _K), jnp.float32),
    )(a2, ub2, sb2, uc2, sc2, ew1, eb1.reshape(1, 1), ew2, eb2.reshape(1, 1))[:_NQ]

    nup = _NU + ((-_NU) % _B)
    au = _padr(all_stu_emb)
    uu = _padr(U_ue[:_NU])
    su = _padr(s_ue[:_NU])
    stu_out = pl.pallas_call(
        _fuse_stu_kernel,
        grid=(nup // _B,),
        in_specs=[
            pl.BlockSpec((_B, _K), lambda i: (i, 0)),
            pl.BlockSpec((_B, _K), lambda i: (i, 0)),
            pl.BlockSpec((_B, 1), lambda i: (i, 0)),
        ],
        out_specs=pl.BlockSpec((_B, _K), lambda i: (i, 0)),
        out_shape=jax.ShapeDtypeStruct((nup, _K), jnp.float32),
    )(au, uu, su)[:_NU]

    return (kn_out, exer_out, stu_out)


# speedup vs baseline: 4.5789x; 4.5789x over previous
import jax
import jax.numpy as jnp
from jax.experimental import pallas as pl
from jax.experimental.pallas import tpu as pltpu

_K = 128
_NQ = 20000
_NU = 10000
_B = 512


def _padr(x, m=_B):
    r = (-x.shape[0]) % m
    if r == 0:
        return x
    return jnp.pad(x, ((0, r),) + ((0, 0),) * (x.ndim - 1))


def _zpq_kernel(h_ref, w_ref, a_ref, z_ref, p_ref, q_ref):
    h = h_ref[...]
    w = w_ref[...]
    at = a_ref[...].T
    z = jnp.dot(h, w.T, preferred_element_type=jnp.float32)
    z_ref[...] = z
    p_ref[...] = jnp.dot(z, at[:_K], preferred_element_type=jnp.float32)
    q_ref[...] = jnp.dot(z, at[_K:], preferred_element_type=jnp.float32)


def _make_edge_a(E, src_lo, dst_lo):
    def k(ed_ref, p_ref, q_ref, e_ref, m_ref):
        pid = pl.program_id(0)

        @pl.when(pid == 0)
        def _():
            m_ref[...] = jnp.full(m_ref.shape, -jnp.inf, jnp.float32)

        def body(i, c):
            g = pid * _B + i
            s = ed_ref[0, i] - src_lo
            d = ed_ref[1, i] - dst_lo
            x = p_ref[pl.ds(s, 1), :] + q_ref[pl.ds(d, 1), :]
            ev = jnp.where(x > 0, x, 0.01 * x)
            ev = jnp.where(g < E, ev, -jnp.inf)
            e_ref[pl.ds(i, 1), :] = ev
            m_ref[pl.ds(d, 1), :] = jnp.maximum(m_ref[pl.ds(d, 1), :], ev)
            return c

        jax.lax.fori_loop(0, _B, body, 0, unroll=8)

    return k


def _make_edge_b(E, src_lo, dst_lo):
    def k(ed_ref, e_ref, m_ref, z_ref, s_ref, u_ref):
        pid = pl.program_id(0)

        @pl.when(pid == 0)
        def _():
            s_ref[...] = jnp.zeros(s_ref.shape, jnp.float32)
            u_ref[...] = jnp.zeros(u_ref.shape, jnp.float32)

        def body(i, c):
            g = pid * _B + i
            s = ed_ref[0, i] - src_lo
            d = ed_ref[1, i] - dst_lo
            ev = e_ref[pl.ds(i, 1), :]
            mv = m_ref[pl.ds(d, 1), :]
            ex = jnp.exp(ev - mv)
            ex = jnp.where(g < E, ex, 0.0)
            s_ref[pl.ds(d, 1), :] = s_ref[pl.ds(d, 1), :] + ex
            u_ref[pl.ds(d, 1), :] = u_ref[pl.ds(d, 1), :] + ex * z_ref[pl.ds(s, 1), :]
            return c

        jax.lax.fori_loop(0, _B, body, 0, unroll=8)

    return k


def _gat(h, edges, W, a, src_lo, src_n, dst_lo, dst_n):
    hp = _padr(h)
    npad = hp.shape[0]
    E = edges.shape[1]
    ep_n = E + ((-E) % _B)
    if ep_n > E:
        pad_col = jnp.array([[src_lo], [dst_lo]], jnp.int32)
        ep = jnp.concatenate([edges, jnp.tile(pad_col, (1, ep_n - E))], axis=1)
    else:
        ep = edges

    z, p, q = pl.pallas_call(
        _zpq_kernel,
        grid=(npad // _B,),
        in_specs=[
            pl.BlockSpec((_B, _K), lambda i: (i, 0)),
            pl.BlockSpec((_K, _K), lambda i: (0, 0)),
            pl.BlockSpec((1, 2 * _K), lambda i: (0, 0)),
        ],
        out_specs=[
            pl.BlockSpec((_B, _K), lambda i: (i, 0)),
            pl.BlockSpec((_B, 1), lambda i: (i, 0)),
            pl.BlockSpec((_B, 1), lambda i: (i, 0)),
        ],
        out_shape=[
            jax.ShapeDtypeStruct((npad, _K), jnp.float32),
            jax.ShapeDtypeStruct((npad, 1), jnp.float32),
            jax.ShapeDtypeStruct((npad, 1), jnp.float32),
        ],
    )(hp, W, a)

    zs = _padr(z[src_lo:src_lo + src_n])
    ps = _padr(p[src_lo:src_lo + src_n])
    qd = _padr(q[dst_lo:dst_lo + dst_n])
    sp = zs.shape[0]
    dp = qd.shape[0]

    e, m = pl.pallas_call(
        _make_edge_a(E, src_lo, dst_lo),
        grid=(ep_n // _B,),
        in_specs=[
            pl.BlockSpec((2, _B), lambda i: (0, i), memory_space=pltpu.SMEM),
            pl.BlockSpec((sp, 1), lambda i: (0, 0)),
            pl.BlockSpec((dp, 1), lambda i: (0, 0)),
        ],
        out_specs=[
            pl.BlockSpec((_B, 1), lambda i: (i, 0)),
            pl.BlockSpec((dp, 1), lambda i: (0, 0)),
        ],
        out_shape=[
            jax.ShapeDtypeStruct((ep_n, 1), jnp.float32),
            jax.ShapeDtypeStruct((dp, 1), jnp.float32),
        ],
    )(ep, ps, qd)

    ssum, U = pl.pallas_call(
        _make_edge_b(E, src_lo, dst_lo),
        grid=(ep_n // _B,),
        in_specs=[
            pl.BlockSpec((2, _B), lambda i: (0, i), memory_space=pltpu.SMEM),
            pl.BlockSpec((_B, 1), lambda i: (i, 0)),
            pl.BlockSpec((dp, 1), lambda i: (0, 0)),
            pl.BlockSpec((sp, _K), lambda i: (0, 0)),
        ],
        out_specs=[
            pl.BlockSpec((dp, 1), lambda i: (0, 0)),
            pl.BlockSpec((dp, _K), lambda i: (0, 0)),
        ],
        out_shape=[
            jax.ShapeDtypeStruct((dp, 1), jnp.float32),
            jax.ShapeDtypeStruct((dp, _K), jnp.float32),
        ],
    )(ep, e, m, zs)
    return U, ssum


def _fuse_kn_kernel(a_ref, ub_ref, sb_ref, uc_ref, sc_ref, ud_ref, sd_ref,
                    w1_ref, b1_ref, w2_ref, b2_ref, w3_ref, b3_ref, o_ref):
    A = a_ref[...]
    B = ub_ref[...] / (sb_ref[...] + 1e-9)
    C = uc_ref[...] / (sc_ref[...] + 1e-9)
    D = ud_ref[...] / (sd_ref[...] + 1e-9)
    w1 = w1_ref[...].T
    w2 = w2_ref[...].T
    w3 = w3_ref[...].T
    s1 = jnp.dot(A, w1[:_K]) + jnp.dot(B, w1[_K:]) + b1_ref[...]
    s2 = jnp.dot(A, w2[:_K]) + jnp.dot(C, w2[_K:]) + b2_ref[...]
    s3 = jnp.dot(A, w3[:_K]) + jnp.dot(D, w3[_K:]) + b3_ref[...]
    mx = jnp.maximum(jnp.maximum(s1, s2), s3)
    e1 = jnp.exp(s1 - mx)
    e2 = jnp.exp(s2 - mx)
    e3 = jnp.exp(s3 - mx)
    den = e1 + e2 + e3
    o_ref[...] = A + (e1 / den) * B + (e2 / den) * C + (e3 / den) * D


def _fuse_exer_kernel(a_ref, ub_ref, sb_ref, uc_ref, sc_ref,
                      w1_ref, b1_ref, w2_ref, b2_ref, o_ref):
    A = a_ref[...]
    B = ub_ref[...] / (sb_ref[...] + 1e-9)
    C = uc_ref[...] / (sc_ref[...] + 1e-9)
    w1 = w1_ref[...].T
    w2 = w2_ref[...].T
    t1 = jnp.dot(A, w1[:_K]) + jnp.dot(B, w1[_K:]) + b1_ref[...]
    t2 = jnp.dot(A, w2[:_K]) + jnp.dot(C, w2[_K:]) + b2_ref[...]
    mx = jnp.maximum(t1, t2)
    e1 = jnp.exp(t1 - mx)
    e2 = jnp.exp(t2 - mx)
    den = e1 + e2
    o_ref[...] = A + (e1 / den) * B + (e2 / den) * C


def _fuse_stu_kernel(a_ref, u_ref, s_ref, o_ref):
    o_ref[...] = a_ref[...] + u_ref[...] / (s_ref[...] + 1e-9)


def kernel(kn_emb, exer_emb, all_stu_emb, dir_edges, undir_edges, ke_edges,
           ek_edges, ue_edges, eu_edges, W_dir, a_dir, W_undir, a_undir,
           W_kfe, a_kfe, W_efk, a_efk, W_ufe, a_ufe, W_efu, a_efu,
           kw1, kb1, kw2, kb2, kw3, kb3, ew1, eb1, ew2, eb2):
    U_dir, s_dir = _gat(kn_emb, dir_edges, W_dir, a_dir, 0, _K, 0, _K)
    U_und, s_und = _gat(kn_emb, undir_edges, W_undir, a_undir, 0, _K, 0, _K)
    e_k = jnp.concatenate([exer_emb, kn_emb], axis=0)
    U_ke, s_ke = _gat(e_k, ke_edges, W_kfe, a_kfe, 0, _NQ, _NQ, _K)
    U_ek, s_ek = _gat(e_k, ek_edges, W_efk, a_efk, _NQ, _K, 0, _NQ)
    e_u = jnp.concatenate([exer_emb, all_stu_emb], axis=0)
    U_ue, s_ue = _gat(e_u, ue_edges, W_ufe, a_ufe, 0, _NQ, _NQ, _NU)
    U_eu, s_eu = _gat(e_u, eu_edges, W_efu, a_efu, _NQ, _NU, 0, _NQ)

    kb1r = kb1.reshape(1, 1)
    kb2r = kb2.reshape(1, 1)
    kb3r = kb3.reshape(1, 1)
    kn_out = pl.pallas_call(
        _fuse_kn_kernel,
        out_shape=jax.ShapeDtypeStruct((_K, _K), jnp.float32),
    )(kn_emb, U_dir[:_K], s_dir[:_K], U_und[:_K], s_und[:_K],
      U_ke[:_K], s_ke[:_K],
      kw1, kb1r, kw2, kb2r, kw3, kb3r)

    nqp = _NQ + ((-_NQ) % _B)
    a2 = _padr(exer_emb)
    ub2 = _padr(U_ek[:_NQ])
    sb2 = _padr(s_ek[:_NQ])
    uc2 = _padr(U_eu[:_NQ])
    sc2 = _padr(s_eu[:_NQ])
    exer_out = pl.pallas_call(
        _fuse_exer_kernel,
        grid=(nqp // _B,),
        in_specs=[
            pl.BlockSpec((_B, _K), lambda i: (i, 0)),
            pl.BlockSpec((_B, _K), lambda i: (i, 0)),
            pl.BlockSpec((_B, 1), lambda i: (i, 0)),
            pl.BlockSpec((_B, _K), lambda i: (i, 0)),
            pl.BlockSpec((_B, 1), lambda i: (i, 0)),
            pl.BlockSpec((1, 2 * _K), lambda i: (0, 0)),
            pl.BlockSpec((1, 1), lambda i: (0, 0)),
            pl.BlockSpec((1, 2 * _K), lambda i: (0, 0)),
            pl.BlockSpec((1, 1), lambda i: (0, 0)),
        ],
        out_specs=pl.BlockSpec((_B, _K), lambda i: (i, 0)),
        out_shape=jax.ShapeDtypeStruct((nqp, _K), jnp.float32),
    )(a2, ub2, sb2, uc2, sc2, ew1, eb1.reshape(1, 1), ew2, eb2.reshape(1, 1))[:_NQ]

    nup = _NU + ((-_NU) % _B)
    au = _padr(all_stu_emb)
    uu = _padr(U_ue[:_NU])
    su = _padr(s_ue[:_NU])
    stu_out = pl.pallas_call(
        _fuse_stu_kernel,
        grid=(nup // _B,),
        in_specs=[
            pl.BlockSpec((_B, _K), lambda i: (i, 0)),
            pl.BlockSpec((_B, _K), lambda i: (i, 0)),
            pl.BlockSpec((_B, 1), lambda i: (i, 0)),
        ],
        out_specs=pl.BlockSpec((_B, _K), lambda i: (i, 0)),
        out_shape=jax.ShapeDtypeStruct((nup, _K), jnp.float32),
    )(au, uu, su)[:_NU]

    return (kn_out, exer_out, stu_out)


# speedup vs baseline: 6.1645x; 1.3463x over previous
import jax
import jax.numpy as jnp
from jax.experimental import pallas as pl
from jax.experimental.pallas import tpu as pltpu

_K = 128
_NQ = 20000
_NU = 10000
_B = 512


def _padr(x, m=_B):
    r = (-x.shape[0]) % m
    if r == 0:
        return x
    return jnp.pad(x, ((0, r),) + ((0, 0),) * (x.ndim - 1))


def _zpq_kernel(h_ref, w_ref, a_ref, z_ref, p_ref, q_ref):
    h = h_ref[...]
    w = w_ref[...]
    at = a_ref[...].T
    z = jnp.dot(h, w.T, preferred_element_type=jnp.float32)
    z_ref[...] = z
    p_ref[...] = jnp.dot(z, at[:_K], preferred_element_type=jnp.float32)
    q_ref[...] = jnp.dot(z, at[_K:], preferred_element_type=jnp.float32)


def _make_edge_a(E, src_lo, dst_lo):
    def k(ed_ref, p_ref, q_ref, e_ref, m_ref):
        pid = pl.program_id(0)

        @pl.when(pid == 0)
        def _():
            m_ref[...] = jnp.full(m_ref.shape, -jnp.inf, jnp.float32)

        def body(i, c):
            g = pid * _B + i
            s = ed_ref[0, i] - src_lo
            d = ed_ref[1, i] - dst_lo
            x = p_ref[pl.ds(s, 1), :] + q_ref[pl.ds(d, 1), :]
            ev = jnp.where(x > 0, x, 0.01 * x)
            ev = jnp.where(g < E, ev, -jnp.inf)
            e_ref[pl.ds(i, 1), :] = ev
            m_ref[pl.ds(d, 1), :] = jnp.maximum(m_ref[pl.ds(d, 1), :], ev)
            return c

        jax.lax.fori_loop(0, _B, body, 0, unroll=16)

    return k


def _make_edge_b(E, src_lo, dst_lo):
    def k(ed_ref, e_ref, m_ref, z_ref, s_ref, u_ref):
        pid = pl.program_id(0)

        @pl.when(pid == 0)
        def _():
            s_ref[...] = jnp.zeros(s_ref.shape, jnp.float32)
            u_ref[...] = jnp.zeros(u_ref.shape, jnp.float32)

        def body(i, c):
            g = pid * _B + i
            s = ed_ref[0, i] - src_lo
            d = ed_ref[1, i] - dst_lo
            ev = e_ref[pl.ds(i, 1), :]
            mv = m_ref[pl.ds(d, 1), :]
            ex = jnp.exp(ev - mv)
            ex = jnp.where(g < E, ex, 0.0)
            s_ref[pl.ds(d, 1), :] = s_ref[pl.ds(d, 1), :] + ex
            u_ref[pl.ds(d, 1), :] = u_ref[pl.ds(d, 1), :] + ex * z_ref[pl.ds(s, 1), :]
            return c

        jax.lax.fori_loop(0, _B, body, 0, unroll=16)

    return k


def _gat(h, edges, W, a, src_lo, src_n, dst_lo, dst_n):
    hp = _padr(h)
    npad = hp.shape[0]
    E = edges.shape[1]
    ep_n = E + ((-E) % _B)
    if ep_n > E:
        pad_col = jnp.array([[src_lo], [dst_lo]], jnp.int32)
        ep = jnp.concatenate([edges, jnp.tile(pad_col, (1, ep_n - E))], axis=1)
    else:
        ep = edges

    z, p, q = pl.pallas_call(
        _zpq_kernel,
        grid=(npad // _B,),
        in_specs=[
            pl.BlockSpec((_B, _K), lambda i: (i, 0)),
            pl.BlockSpec((_K, _K), lambda i: (0, 0)),
            pl.BlockSpec((1, 2 * _K), lambda i: (0, 0)),
        ],
        out_specs=[
            pl.BlockSpec((_B, _K), lambda i: (i, 0)),
            pl.BlockSpec((_B, 1), lambda i: (i, 0)),
            pl.BlockSpec((_B, 1), lambda i: (i, 0)),
        ],
        out_shape=[
            jax.ShapeDtypeStruct((npad, _K), jnp.float32),
            jax.ShapeDtypeStruct((npad, 1), jnp.float32),
            jax.ShapeDtypeStruct((npad, 1), jnp.float32),
        ],
    )(hp, W, a)

    zs = _padr(z[src_lo:src_lo + src_n])
    ps = _padr(p[src_lo:src_lo + src_n])
    qd = _padr(q[dst_lo:dst_lo + dst_n])
    sp = zs.shape[0]
    dp = qd.shape[0]

    e, m = pl.pallas_call(
        _make_edge_a(E, src_lo, dst_lo),
        grid=(ep_n // _B,),
        in_specs=[
            pl.BlockSpec((2, _B), lambda i: (0, i), memory_space=pltpu.SMEM),
            pl.BlockSpec((sp, 1), lambda i: (0, 0)),
            pl.BlockSpec((dp, 1), lambda i: (0, 0)),
        ],
        out_specs=[
            pl.BlockSpec((_B, 1), lambda i: (i, 0)),
            pl.BlockSpec((dp, 1), lambda i: (0, 0)),
        ],
        out_shape=[
            jax.ShapeDtypeStruct((ep_n, 1), jnp.float32),
            jax.ShapeDtypeStruct((dp, 1), jnp.float32),
        ],
    )(ep, ps, qd)

    ssum, U = pl.pallas_call(
        _make_edge_b(E, src_lo, dst_lo),
        grid=(ep_n // _B,),
        in_specs=[
            pl.BlockSpec((2, _B), lambda i: (0, i), memory_space=pltpu.SMEM),
            pl.BlockSpec((_B, 1), lambda i: (i, 0)),
            pl.BlockSpec((dp, 1), lambda i: (0, 0)),
            pl.BlockSpec((sp, _K), lambda i: (0, 0)),
        ],
        out_specs=[
            pl.BlockSpec((dp, 1), lambda i: (0, 0)),
            pl.BlockSpec((dp, _K), lambda i: (0, 0)),
        ],
        out_shape=[
            jax.ShapeDtypeStruct((dp, 1), jnp.float32),
            jax.ShapeDtypeStruct((dp, _K), jnp.float32),
        ],
    )(ep, e, m, zs)
    return U, ssum


def _fuse_kn_kernel(a_ref, ub_ref, sb_ref, uc_ref, sc_ref, ud_ref, sd_ref,
                    w1_ref, b1_ref, w2_ref, b2_ref, w3_ref, b3_ref, o_ref):
    A = a_ref[...]
    B = ub_ref[...] / (sb_ref[...] + 1e-9)
    C = uc_ref[...] / (sc_ref[...] + 1e-9)
    D = ud_ref[...] / (sd_ref[...] + 1e-9)
    w1 = w1_ref[...].T
    w2 = w2_ref[...].T
    w3 = w3_ref[...].T
    s1 = jnp.dot(A, w1[:_K]) + jnp.dot(B, w1[_K:]) + b1_ref[...]
    s2 = jnp.dot(A, w2[:_K]) + jnp.dot(C, w2[_K:]) + b2_ref[...]
    s3 = jnp.dot(A, w3[:_K]) + jnp.dot(D, w3[_K:]) + b3_ref[...]
    mx = jnp.maximum(jnp.maximum(s1, s2), s3)
    e1 = jnp.exp(s1 - mx)
    e2 = jnp.exp(s2 - mx)
    e3 = jnp.exp(s3 - mx)
    den = e1 + e2 + e3
    o_ref[...] = A + (e1 / den) * B + (e2 / den) * C + (e3 / den) * D


def _fuse_exer_kernel(a_ref, ub_ref, sb_ref, uc_ref, sc_ref,
                      w1_ref, b1_ref, w2_ref, b2_ref, o_ref):
    A = a_ref[...]
    B = ub_ref[...] / (sb_ref[...] + 1e-9)
    C = uc_ref[...] / (sc_ref[...] + 1e-9)
    w1 = w1_ref[...].T
    w2 = w2_ref[...].T
    t1 = jnp.dot(A, w1[:_K]) + jnp.dot(B, w1[_K:]) + b1_ref[...]
    t2 = jnp.dot(A, w2[:_K]) + jnp.dot(C, w2[_K:]) + b2_ref[...]
    mx = jnp.maximum(t1, t2)
    e1 = jnp.exp(t1 - mx)
    e2 = jnp.exp(t2 - mx)
    den = e1 + e2
    o_ref[...] = A + (e1 / den) * B + (e2 / den) * C


def _fuse_stu_kernel(a_ref, u_ref, s_ref, o_ref):
    o_ref[...] = a_ref[...] + u_ref[...] / (s_ref[...] + 1e-9)


def kernel(kn_emb, exer_emb, all_stu_emb, dir_edges, undir_edges, ke_edges,
           ek_edges, ue_edges, eu_edges, W_dir, a_dir, W_undir, a_undir,
           W_kfe, a_kfe, W_efk, a_efk, W_ufe, a_ufe, W_efu, a_efu,
           kw1, kb1, kw2, kb2, kw3, kb3, ew1, eb1, ew2, eb2):
    U_dir, s_dir = _gat(kn_emb, dir_edges, W_dir, a_dir, 0, _K, 0, _K)
    U_und, s_und = _gat(kn_emb, undir_edges, W_undir, a_undir, 0, _K, 0, _K)
    e_k = jnp.concatenate([exer_emb, kn_emb], axis=0)
    U_ke, s_ke = _gat(e_k, ke_edges, W_kfe, a_kfe, 0, _NQ, _NQ, _K)
    U_ek, s_ek = _gat(e_k, ek_edges, W_efk, a_efk, _NQ, _K, 0, _NQ)
    e_u = jnp.concatenate([exer_emb, all_stu_emb], axis=0)
    U_ue, s_ue = _gat(e_u, ue_edges, W_ufe, a_ufe, 0, _NQ, _NQ, _NU)
    U_eu, s_eu = _gat(e_u, eu_edges, W_efu, a_efu, _NQ, _NU, 0, _NQ)

    kb1r = kb1.reshape(1, 1)
    kb2r = kb2.reshape(1, 1)
    kb3r = kb3.reshape(1, 1)
    kn_out = pl.pallas_call(
        _fuse_kn_kernel,
        out_shape=jax.ShapeDtypeStruct((_K, _K), jnp.float32),
    )(kn_emb, U_dir[:_K], s_dir[:_K], U_und[:_K], s_und[:_K],
      U_ke[:_K], s_ke[:_K],
      kw1, kb1r, kw2, kb2r, kw3, kb3r)

    nqp = _NQ + ((-_NQ) % _B)
    a2 = _padr(exer_emb)
    ub2 = _padr(U_ek[:_NQ])
    sb2 = _padr(s_ek[:_NQ])
    uc2 = _padr(U_eu[:_NQ])
    sc2 = _padr(s_eu[:_NQ])
    exer_out = pl.pallas_call(
        _fuse_exer_kernel,
        grid=(nqp // _B,),
        in_specs=[
            pl.BlockSpec((_B, _K), lambda i: (i, 0)),
            pl.BlockSpec((_B, _K), lambda i: (i, 0)),
            pl.BlockSpec((_B, 1), lambda i: (i, 0)),
            pl.BlockSpec((_B, _K), lambda i: (i, 0)),
            pl.BlockSpec((_B, 1), lambda i: (i, 0)),
            pl.BlockSpec((1, 2 * _K), lambda i: (0, 0)),
            pl.BlockSpec((1, 1), lambda i: (0, 0)),
            pl.BlockSpec((1, 2 * _K), lambda i: (0, 0)),
            pl.BlockSpec((1, 1), lambda i: (0, 0)),
        ],
        out_specs=pl.BlockSpec((_B, _K), lambda i: (i, 0)),
        out_shape=jax.ShapeDtypeStruct((nqp, _K), jnp.float32),
    )(a2, ub2, sb2, uc2, sc2, ew1, eb1.reshape(1, 1), ew2, eb2.reshape(1, 1))[:_NQ]

    nup = _NU + ((-_NU) % _B)
    au = _padr(all_stu_emb)
    uu = _padr(U_ue[:_NU])
    su = _padr(s_ue[:_NU])
    stu_out = pl.pallas_call(
        _fuse_stu_kernel,
        grid=(nup // _B,),
        in_specs=[
            pl.BlockSpec((_B, _K), lambda i: (i, 0)),
            pl.BlockSpec((_B, _K), lambda i: (i, 0)),
            pl.BlockSpec((_B, 1), lambda i: (i, 0)),
        ],
        out_specs=pl.BlockSpec((_B, _K), lambda i: (i, 0)),
        out_shape=jax.ShapeDtypeStruct((nup, _K), jnp.float32),
    )(au, uu, su)[:_NU]

    return (kn_out, exer_out, stu_out)
